# baseline (speedup 1.0000x reference)
import functools

import jax
import jax.numpy as jnp
from jax import lax
from jax.experimental import pallas as pl
from jax.experimental.pallas import tpu as pltpu
from jax.experimental.pallas import tpu_sc as plsc

N = 10000
E = 160000
FEAT = 256
EMB = 256
HALF = EMB // 2

NC, NS, L = 2, 16, 16
CH = 128
EP = 163840
NCHUNK = EP // CH
CPT = NCHUNK // NS
NBUF = 2
GRP = 8
NP = 10240
DEG_ROWS = NP // NS
ACC_ROWS = NP // NS

_mesh = functools.partial(
    plsc.VectorSubcoreMesh, core_axis_name="c", subcore_axis_name="s"
)


def _deg_body(dst_hbm, deg_hbm, ones_v, dst_all, deg_sh, sem):
    c = lax.axis_index("c")
    s = lax.axis_index("s")

    @pl.when(c == 0)
    def _():
        for i in range(128 // L):
            ones_v[pl.ds(i * L, L)] = jnp.ones((L,), jnp.float32)
        pltpu.sync_copy(dst_hbm.at[pl.ds(s * CPT, CPT)], dst_all)
        base = s * DEG_ROWS
        for j in range(DEG_ROWS // 128):
            pltpu.sync_copy(ones_v, deg_sh.at[pl.ds(base + j * 128, 128)])
        plsc.subcore_barrier()

        ones_c = ones_v.at[pl.ds(0, CH)]

        def group(g, carry):
            for j in range(GRP):
                pltpu.async_copy(
                    ones_c, deg_sh.at[dst_all.at[g * GRP + j]], sem, add=True
                )
            for j in range(GRP):
                pltpu.make_async_copy(
                    ones_c, deg_sh.at[dst_all.at[g * GRP + j]], sem
                ).wait()
            return carry

        lax.fori_loop(0, CPT // GRP, group, 0)
        plsc.subcore_barrier()
        pltpu.sync_copy(
            deg_sh.at[pl.ds(base, DEG_ROWS)], deg_hbm.at[pl.ds(base, DEG_ROWS)]
        )


def _degree(dst2):
    run = pl.kernel(
        _deg_body,
        out_type=jax.ShapeDtypeStruct((NP,), jnp.float32),
        mesh=_mesh(),
        scratch_types=[
            pltpu.VMEM((128,), jnp.float32),
            pltpu.VMEM((CPT, CH), jnp.int32),
            pltpu.VMEM_SHARED((NP,), jnp.float32),
            pltpu.SemaphoreType.DMA,
        ],
    )
    return run(dst2)


def _gemm_body(x_ref, w_ref, deg_ref, h0_ref, h1_ref):
    h = lax.dot_general(
        x_ref[...], w_ref[...], (((1,), (1,)), ((), ())),
        preferred_element_type=jnp.float32,
    )
    h = h * lax.rsqrt(deg_ref[...])
    h0_ref[...] = h[:, :HALF]
    h1_ref[...] = h[:, HALF:]


def _gemm_scaled(x, W, deg):
    R = 1024
    return pl.pallas_call(
        _gemm_body,
        grid=(NP // R,),
        in_specs=[
            pl.BlockSpec((R, FEAT), lambda i: (i, 0)),
            pl.BlockSpec((EMB, FEAT), lambda i: (0, 0)),
            pl.BlockSpec((R, 1), lambda i: (i, 0)),
        ],
        out_specs=[
            pl.BlockSpec((R, HALF), lambda i: (i, 0)),
            pl.BlockSpec((R, HALF), lambda i: (i, 0)),
        ],
        out_shape=[jax.ShapeDtypeStruct((NP, HALF), jnp.float32)] * 2,
    )(x, W, deg)


def _spmm_body(src_hbm, dst_hbm, h0_hbm, h1_hbm, a0_hbm, a1_hbm,
               idx_all, dst_all, rows, acc_sh):
    c = lax.axis_index("c")
    s = lax.axis_index("s")
    base = s * ACC_ROWS

    def half(h_hbm, out_hbm):
        pltpu.sync_copy(src_hbm.at[pl.ds(s * CPT * CH, CPT * CH)], idx_all)
        pltpu.sync_copy(dst_hbm.at[pl.ds(s * CPT, CPT)], dst_all)
        pltpu.sync_copy(
            h_hbm.at[pl.ds(base, ACC_ROWS)], acc_sh.at[pl.ds(base, ACC_ROWS)]
        )
        plsc.subcore_barrier()

        def chunk(k, carry):
            pltpu.sync_copy(h_hbm.at[idx_all.at[pl.ds(k * CH, CH)]], rows)
            pltpu.sync_copy(rows, acc_sh.at[dst_all.at[k]], add=True)
            return carry

        lax.fori_loop(0, CPT, chunk, 0)

        plsc.subcore_barrier()
        pltpu.sync_copy(
            acc_sh.at[pl.ds(base, ACC_ROWS)], out_hbm.at[pl.ds(base, ACC_ROWS)]
        )

    @pl.when(c == 0)
    def _():
        half(h0_hbm, a0_hbm)

    @pl.when(c == 1)
    def _():
        half(h1_hbm, a1_hbm)


def _spmm(src2, dst2, h0, h1):
    run = pl.kernel(
        _spmm_body,
        out_type=[jax.ShapeDtypeStruct((NP, HALF), jnp.float32)] * 2,
        mesh=_mesh(),
        scratch_types=[
            pltpu.VMEM((CPT * CH,), jnp.int32),
            pltpu.VMEM((CPT, CH), jnp.int32),
            pltpu.VMEM((CH, HALF), jnp.float32),
            pltpu.VMEM_SHARED((NP, HALF), jnp.float32),
        ],
    )
    return run(src2, dst2, h0, h1)


def _combine_body(a0_ref, a1_ref, deg_ref, out_ref):
    isd = lax.rsqrt(deg_ref[...])
    out_ref[:, :HALF] = a0_ref[...] * isd
    out_ref[:, HALF:] = a1_ref[...] * isd


def _combine(a0, a1, deg):
    R = 1000
    return pl.pallas_call(
        _combine_body,
        grid=(N // R,),
        in_specs=[
            pl.BlockSpec((R, HALF), lambda i: (i, 0)),
            pl.BlockSpec((R, HALF), lambda i: (i, 0)),
            pl.BlockSpec((R, 1), lambda i: (i, 0)),
        ],
        out_specs=pl.BlockSpec((R, EMB), lambda i: (i, 0)),
        out_shape=jax.ShapeDtypeStruct((N, EMB), jnp.float32),
    )(a0, a1, deg)


def kernel(x, edge_index, W):
    src = edge_index[0].astype(jnp.int32)
    dst = edge_index[1].astype(jnp.int32)
    pad = EP - E
    src1 = jnp.concatenate([src, jnp.zeros((pad,), jnp.int32)])
    dst2 = jnp.concatenate(
        [dst, jnp.full((pad,), NP - 1, jnp.int32)]).reshape(NCHUNK, CH)
    deg = _degree(dst2).reshape(NP, 1)
    h0, h1 = _gemm_scaled(x, W, deg)
    a0, a1 = _spmm(src1, dst2, h0, h1)
    return _combine(a0, a1, deg)


# speedup vs baseline: 1.2733x; 1.2733x over previous
import functools

import jax
import jax.numpy as jnp
from jax import lax
from jax.experimental import pallas as pl
from jax.experimental.pallas import tpu as pltpu
from jax.experimental.pallas import tpu_sc as plsc

N = 10000
E = 160000
FEAT = 256
EMB = 256
HALF = EMB // 2

NC, NS, L = 2, 16, 16
CH = 128
EP = 163840
NCHUNK = EP // CH
CPT = NCHUNK // NS
NBUF = 2
GRP = 8
NP = 10240
DEG_ROWS = NP // NS
ACC_ROWS = NP // NS

_mesh = functools.partial(
    plsc.VectorSubcoreMesh, core_axis_name="c", subcore_axis_name="s"
)


def _deg_body(dst_hbm, deg_hbm, ones_v, dst_all, deg_sh, sem):
    c = lax.axis_index("c")
    s = lax.axis_index("s")

    @pl.when(c == 0)
    def _():
        for i in range(128 // L):
            ones_v[pl.ds(i * L, L)] = jnp.ones((L,), jnp.float32)
        pltpu.sync_copy(dst_hbm.at[pl.ds(s * CPT, CPT)], dst_all)
        base = s * DEG_ROWS
        for j in range(DEG_ROWS // 128):
            pltpu.sync_copy(ones_v, deg_sh.at[pl.ds(base + j * 128, 128)])
        plsc.subcore_barrier()

        ones_c = ones_v.at[pl.ds(0, CH)]

        def group(g, carry):
            for j in range(GRP):
                pltpu.async_copy(
                    ones_c, deg_sh.at[dst_all.at[g * GRP + j]], sem, add=True
                )
            for j in range(GRP):
                pltpu.make_async_copy(
                    ones_c, deg_sh.at[dst_all.at[g * GRP + j]], sem
                ).wait()
            return carry

        lax.fori_loop(0, CPT // GRP, group, 0)
        plsc.subcore_barrier()
        pltpu.sync_copy(
            deg_sh.at[pl.ds(base, DEG_ROWS)], deg_hbm.at[pl.ds(base, DEG_ROWS)]
        )


def _degree(dst2):
    run = pl.kernel(
        _deg_body,
        out_type=jax.ShapeDtypeStruct((NP,), jnp.float32),
        mesh=_mesh(),
        scratch_types=[
            pltpu.VMEM((128,), jnp.float32),
            pltpu.VMEM((CPT, CH), jnp.int32),
            pltpu.VMEM_SHARED((NP,), jnp.float32),
            pltpu.SemaphoreType.DMA,
        ],
    )
    return run(dst2)


def _gemm_body(x_ref, w_ref, deg_ref, h0_ref, h1_ref):
    h = lax.dot_general(
        x_ref[...], w_ref[...], (((1,), (1,)), ((), ())),
        preferred_element_type=jnp.float32,
    )
    h = h * lax.rsqrt(deg_ref[...])
    h0_ref[...] = h[:, :HALF]
    h1_ref[...] = h[:, HALF:]


def _gemm_scaled(x, W, deg):
    R = 1024
    return pl.pallas_call(
        _gemm_body,
        grid=(NP // R,),
        in_specs=[
            pl.BlockSpec((R, FEAT), lambda i: (i, 0)),
            pl.BlockSpec((EMB, FEAT), lambda i: (0, 0)),
            pl.BlockSpec((R, 1), lambda i: (i, 0)),
        ],
        out_specs=[
            pl.BlockSpec((R, HALF), lambda i: (i, 0)),
            pl.BlockSpec((R, HALF), lambda i: (i, 0)),
        ],
        out_shape=[jax.ShapeDtypeStruct((NP, HALF), jnp.float32)] * 2,
    )(x, W, deg)


def _spmm_body(src_hbm, dst_hbm, h0_hbm, h1_hbm, a0_hbm, a1_hbm,
               idx2, dst2, rows, acc_sh, sem_i, sem_d, sem_g, sem_s):
    c = lax.axis_index("c")
    s = lax.axis_index("s")
    base = s * ACC_ROWS

    def off(j):
        return (j * NS + s) * CH

    def half(h_hbm, out_hbm):
        def stage_start(j, p):
            pltpu.async_copy(src_hbm.at[pl.ds(off(j), CH)], idx2.at[p],
                             sem_i.at[p])
            pltpu.async_copy(dst_hbm.at[pl.ds(off(j), CH)], dst2.at[p],
                             sem_d.at[p])

        def stage_wait(j, p):
            pltpu.make_async_copy(src_hbm.at[pl.ds(off(j), CH)], idx2.at[p],
                                  sem_i.at[p]).wait()
            pltpu.make_async_copy(dst_hbm.at[pl.ds(off(j), CH)], dst2.at[p],
                                  sem_d.at[p]).wait()

        def gather_start(p):
            pltpu.async_copy(h_hbm.at[idx2.at[p]], rows.at[p], sem_g.at[p])

        def gather_wait(p):
            pltpu.make_async_copy(
                h_hbm.at[idx2.at[p]], rows.at[p], sem_g.at[p]
            ).wait()

        def scatter_start(p):
            pltpu.async_copy(
                rows.at[p], acc_sh.at[dst2.at[p]], sem_s.at[p], add=True
            )

        def scatter_wait(p):
            pltpu.make_async_copy(
                rows.at[p], acc_sh.at[dst2.at[p]], sem_s.at[p]
            ).wait()

        stage_start(0, 0)
        pltpu.sync_copy(
            h_hbm.at[pl.ds(base, ACC_ROWS)], acc_sh.at[pl.ds(base, ACC_ROWS)]
        )
        plsc.subcore_barrier()

        def pair(q, carry):
            for p in range(NBUF):
                j = q * NBUF + p
                stage_wait(j, p)
                gather_start(p)

                @pl.when(j >= 1)
                def _():
                    scatter_wait(1 - p)

                @pl.when(j + 1 < CPT)
                def _():
                    stage_start(j + 1, 1 - p)

                gather_wait(p)
                scatter_start(p)

            return carry

        lax.fori_loop(0, CPT // NBUF, pair, 0)
        scatter_wait((CPT - 1) % NBUF)

        plsc.subcore_barrier()
        pltpu.sync_copy(
            acc_sh.at[pl.ds(base, ACC_ROWS)], out_hbm.at[pl.ds(base, ACC_ROWS)]
        )

    @pl.when(c == 0)
    def _():
        half(h0_hbm, a0_hbm)

    @pl.when(c == 1)
    def _():
        half(h1_hbm, a1_hbm)


def _spmm(src1, dst1, h0, h1):
    run = pl.kernel(
        _spmm_body,
        out_type=[jax.ShapeDtypeStruct((NP, HALF), jnp.float32)] * 2,
        mesh=_mesh(),
        scratch_types=[
            pltpu.VMEM((NBUF, CH), jnp.int32),
            pltpu.VMEM((NBUF, CH), jnp.int32),
            pltpu.VMEM((NBUF, CH, HALF), jnp.float32),
            pltpu.VMEM_SHARED((NP, HALF), jnp.float32),
            pltpu.SemaphoreType.DMA((NBUF,)),
            pltpu.SemaphoreType.DMA((NBUF,)),
            pltpu.SemaphoreType.DMA((NBUF,)),
            pltpu.SemaphoreType.DMA((NBUF,)),
        ],
    )
    return run(src1, dst1, h0, h1)


def _combine_body(a0_ref, a1_ref, deg_ref, out_ref):
    isd = lax.rsqrt(deg_ref[...])
    out_ref[:, :HALF] = a0_ref[...] * isd
    out_ref[:, HALF:] = a1_ref[...] * isd


def _combine(a0, a1, deg):
    R = 1000
    return pl.pallas_call(
        _combine_body,
        grid=(N // R,),
        in_specs=[
            pl.BlockSpec((R, HALF), lambda i: (i, 0)),
            pl.BlockSpec((R, HALF), lambda i: (i, 0)),
            pl.BlockSpec((R, 1), lambda i: (i, 0)),
        ],
        out_specs=pl.BlockSpec((R, EMB), lambda i: (i, 0)),
        out_shape=jax.ShapeDtypeStruct((N, EMB), jnp.float32),
    )(a0, a1, deg)


def kernel(x, edge_index, W):
    src = edge_index[0].astype(jnp.int32)
    dst = edge_index[1].astype(jnp.int32)
    pad = EP - E
    src1 = jnp.concatenate([src, jnp.zeros((pad,), jnp.int32)])
    dst1 = jnp.concatenate([dst, jnp.full((pad,), NP - 1, jnp.int32)])
    deg = _degree(dst1.reshape(NCHUNK, CH)).reshape(NP, 1)
    h0, h1 = _gemm_scaled(x, W, deg)
    a0, a1 = _spmm(src1, dst1, h0, h1)
    return _combine(a0, a1, deg)


# speedup vs baseline: 1.2763x; 1.0024x over previous
import functools

import jax
import jax.numpy as jnp
from jax import lax
from jax.experimental import pallas as pl
from jax.experimental.pallas import tpu as pltpu
from jax.experimental.pallas import tpu_sc as plsc

N = 10000
E = 160000
FEAT = 256
EMB = 256
HALF = EMB // 2

NC, NS, L = 2, 16, 16
CH = 128
EP = 163840
NCHUNK = EP // CH
CPT = NCHUNK // NS
NBUF = 2
GRP = 8
NP = 10240
DEG_ROWS = NP // NS
ACC_ROWS = NP // NS

_mesh = functools.partial(
    plsc.VectorSubcoreMesh, core_axis_name="c", subcore_axis_name="s"
)


def _deg_body(dst_hbm, deg_hbm, ones_v, dst_all, deg_sh, sem):
    c = lax.axis_index("c")
    s = lax.axis_index("s")

    @pl.when(c == 0)
    def _():
        for i in range(128 // L):
            ones_v[pl.ds(i * L, L)] = jnp.ones((L,), jnp.float32)
        pltpu.sync_copy(dst_hbm.at[pl.ds(s * CPT, CPT)], dst_all)
        base = s * DEG_ROWS
        for j in range(DEG_ROWS // 128):
            pltpu.sync_copy(ones_v, deg_sh.at[pl.ds(base + j * 128, 128)])
        plsc.subcore_barrier()

        ones_c = ones_v.at[pl.ds(0, CH)]

        def group(g, carry):
            for j in range(GRP):
                pltpu.async_copy(
                    ones_c, deg_sh.at[dst_all.at[g * GRP + j]], sem, add=True
                )
            for j in range(GRP):
                pltpu.make_async_copy(
                    ones_c, deg_sh.at[dst_all.at[g * GRP + j]], sem
                ).wait()
            return carry

        lax.fori_loop(0, CPT // GRP, group, 0)
        plsc.subcore_barrier()
        pltpu.sync_copy(
            deg_sh.at[pl.ds(base, DEG_ROWS)], deg_hbm.at[pl.ds(base, DEG_ROWS)]
        )


def _degree(dst2):
    run = pl.kernel(
        _deg_body,
        out_type=jax.ShapeDtypeStruct((NP,), jnp.float32),
        mesh=_mesh(),
        scratch_types=[
            pltpu.VMEM((128,), jnp.float32),
            pltpu.VMEM((CPT, CH), jnp.int32),
            pltpu.VMEM_SHARED((NP,), jnp.float32),
            pltpu.SemaphoreType.DMA,
        ],
    )
    return run(dst2)


def _gemm_body(x_ref, w_ref, deg_ref, h0_ref, h1_ref):
    h = lax.dot_general(
        x_ref[...], w_ref[...], (((1,), (1,)), ((), ())),
        preferred_element_type=jnp.float32,
    )
    h = h * lax.rsqrt(deg_ref[...])
    h0_ref[...] = h[:, :HALF]
    h1_ref[...] = h[:, HALF:]


def _gemm_scaled(x, W, deg):
    R = 1024
    return pl.pallas_call(
        _gemm_body,
        grid=(NP // R,),
        in_specs=[
            pl.BlockSpec((R, FEAT), lambda i: (i, 0)),
            pl.BlockSpec((EMB, FEAT), lambda i: (0, 0)),
            pl.BlockSpec((R, 1), lambda i: (i, 0)),
        ],
        out_specs=[
            pl.BlockSpec((R, HALF), lambda i: (i, 0)),
            pl.BlockSpec((R, HALF), lambda i: (i, 0)),
        ],
        out_shape=[jax.ShapeDtypeStruct((NP, HALF), jnp.float32)] * 2,
    )(x, W, deg)


def _spmm_body(ed_hbm, h0_hbm, h1_hbm, a0_hbm, a1_hbm,
               ed2, rows, acc_sh, sem_i, sem_g, sem_s):
    c = lax.axis_index("c")
    s = lax.axis_index("s")
    base = s * ACC_ROWS

    def half(h_hbm, out_hbm):
        def stage_start(j, p):
            pltpu.async_copy(ed_hbm.at[pl.ds(j * NS + s, 1)],
                             ed2.at[pl.ds(p, 1)], sem_i.at[p])

        def stage_wait(j, p):
            pltpu.make_async_copy(ed_hbm.at[pl.ds(j * NS + s, 1)],
                                  ed2.at[pl.ds(p, 1)], sem_i.at[p]).wait()

        def gather_start(p):
            pltpu.async_copy(h_hbm.at[ed2.at[p, 0]], rows.at[p], sem_g.at[p])

        def gather_wait(p):
            pltpu.make_async_copy(
                h_hbm.at[ed2.at[p, 0]], rows.at[p], sem_g.at[p]
            ).wait()

        def scatter_start(p):
            pltpu.async_copy(
                rows.at[p], acc_sh.at[ed2.at[p, 1]], sem_s.at[p], add=True
            )

        def scatter_wait(p):
            pltpu.make_async_copy(
                rows.at[p], acc_sh.at[ed2.at[p, 1]], sem_s.at[p]
            ).wait()

        stage_start(0, 0)
        pltpu.sync_copy(
            h_hbm.at[pl.ds(base, ACC_ROWS)], acc_sh.at[pl.ds(base, ACC_ROWS)]
        )
        plsc.subcore_barrier()

        def pair(q, carry):
            for p in range(NBUF):
                j = q * NBUF + p
                stage_wait(j, p)
                gather_start(p)

                @pl.when(j >= 1)
                def _():
                    scatter_wait(1 - p)

                @pl.when(j + 1 < CPT)
                def _():
                    stage_start(j + 1, 1 - p)

                gather_wait(p)
                scatter_start(p)

            return carry

        lax.fori_loop(0, CPT // NBUF, pair, 0)
        scatter_wait((CPT - 1) % NBUF)

        plsc.subcore_barrier()
        pltpu.sync_copy(
            acc_sh.at[pl.ds(base, ACC_ROWS)], out_hbm.at[pl.ds(base, ACC_ROWS)]
        )

    @pl.when(c == 0)
    def _():
        half(h0_hbm, a0_hbm)

    @pl.when(c == 1)
    def _():
        half(h1_hbm, a1_hbm)


def _spmm(ed3, h0, h1):
    run = pl.kernel(
        _spmm_body,
        out_type=[jax.ShapeDtypeStruct((NP, HALF), jnp.float32)] * 2,
        mesh=_mesh(),
        scratch_types=[
            pltpu.VMEM((NBUF, 2, CH), jnp.int32),
            pltpu.VMEM((NBUF, CH, HALF), jnp.float32),
            pltpu.VMEM_SHARED((NP, HALF), jnp.float32),
            pltpu.SemaphoreType.DMA((NBUF,)),
            pltpu.SemaphoreType.DMA((NBUF,)),
            pltpu.SemaphoreType.DMA((NBUF,)),
        ],
    )
    return run(ed3, h0, h1)


def _combine_body(a0_ref, a1_ref, deg_ref, out_ref):
    isd = lax.rsqrt(deg_ref[...])
    out_ref[:, :HALF] = a0_ref[...] * isd
    out_ref[:, HALF:] = a1_ref[...] * isd


def _combine(a0, a1, deg):
    R = 1000
    return pl.pallas_call(
        _combine_body,
        grid=(N // R,),
        in_specs=[
            pl.BlockSpec((R, HALF), lambda i: (i, 0)),
            pl.BlockSpec((R, HALF), lambda i: (i, 0)),
            pl.BlockSpec((R, 1), lambda i: (i, 0)),
        ],
        out_specs=pl.BlockSpec((R, EMB), lambda i: (i, 0)),
        out_shape=jax.ShapeDtypeStruct((N, EMB), jnp.float32),
    )(a0, a1, deg)


def kernel(x, edge_index, W):
    src = edge_index[0].astype(jnp.int32)
    dst = edge_index[1].astype(jnp.int32)
    pad = EP - E
    src1 = jnp.concatenate([src, jnp.zeros((pad,), jnp.int32)])
    dst1 = jnp.concatenate([dst, jnp.full((pad,), NP - 1, jnp.int32)])
    ed3 = jnp.stack(
        [src1.reshape(NCHUNK, CH), dst1.reshape(NCHUNK, CH)], axis=1)
    deg = _degree(dst1.reshape(NCHUNK, CH)).reshape(NP, 1)
    h0, h1 = _gemm_scaled(x, W, deg)
    a0, a1 = _spmm(ed3, h0, h1)
    return _combine(a0, a1, deg)
